# baseline (speedup 1.0000x reference)
import functools

import jax
import jax.numpy as jnp
from jax import lax
from jax.experimental import pallas as pl
from jax.experimental.pallas import tpu as pltpu
from jax.experimental.pallas import tpu_sc as plsc

_B = 16384
_EMB = 16
_NC = 2
_NS = 16
_NW = _NC * _NS

_mesh = plsc.VectorSubcoreMesh(core_axis_name="c", subcore_axis_name="s")


def _make_sc_gather(b):
    bpw = b // _NW
    ngrp = bpw // 16
    fbuf = min(256, bpw)
    fmask = fbuf // 16 - 1

    @functools.partial(
        pl.kernel,
        mesh=_mesh,
        out_type=[jax.ShapeDtypeStruct((b, _EMB), jnp.float32)] * 4,
        scratch_types=[
            pltpu.VMEM((bpw,), jnp.int32),
            pltpu.VMEM((bpw,), jnp.int32),
            [pltpu.VMEM((_EMB, 128), jnp.float32)] * 32,
            pltpu.VMEM((fbuf, _EMB), jnp.float32),
            [pltpu.SemaphoreType.DMA] * 2,
        ],
        compiler_params=pltpu.CompilerParams(needs_layout_passes=False),
    )
    def _sc_gather(uidx_hbm, iidx_hbm, gu_hbm, gi_hbm, mu_hbm, mi_hbm,
                   gu_out, gi_out, mu_out, mi_out,
                   uidx_v, iidx_v, stages, rowbuf, sems):
        wid = lax.axis_index("s") * _NC + lax.axis_index("c")
        base = wid * bpw

        pltpu.sync_copy(uidx_hbm.at[wid], uidx_v)
        pltpu.sync_copy(iidx_hbm.at[wid], iidx_v)

        lanes = lax.iota(jnp.int32, 16)

        for table, idx_v, out in (
            (gu_hbm, uidx_v, gu_out),
            (gi_hbm, iidx_v, gi_out),
            (mu_hbm, uidx_v, mu_out),
            (mi_hbm, iidx_v, mi_out),
        ):
            def fire(g, ring, table=table, idx_v=idx_v):
                vec = idx_v[pl.ds(g * 16, 16)]
                for j in range(16):
                    u = vec[j]
                    bs = pl.multiple_of((u >> 7) * 128, 128)
                    pltpu.async_copy(
                        table.at[:, pl.ds(bs, 128)], stages[ring * 16 + j],
                        sems[ring])

            def extract(g, ring, table=table, idx_v=idx_v):
                for j in range(16):
                    pltpu.make_async_copy(
                        table.at[:, pl.ds(0, 128)], stages[ring * 16 + j],
                        sems[ring]).wait()
                vec = idx_v[pl.ds(g * 16, 16)]
                for j in range(16):
                    c = vec[j] & 127
                    val = plsc.load_gather(
                        stages[ring * 16 + j],
                        [lanes, jnp.zeros((16,), jnp.int32) + c])
                    rowbuf[(g & fmask) * 16 + j, :] = val

            fire(0, 0)
            fire(1, 1)

            def body(h, carry, out=out, fire=fire, extract=extract):
                g0 = 2 * h
                g1 = 2 * h + 1
                extract(g0, 0)

                @pl.when(g0 + 2 < ngrp)
                def _():
                    fire(g0 + 2, 0)

                extract(g1, 1)

                @pl.when(g1 + 2 < ngrp)
                def _():
                    fire(g1 + 2, 1)

                @pl.when((g1 & fmask) == fmask)
                def _():
                    start = pl.multiple_of(
                        base + (g1 // (fbuf // 16)) * fbuf, fbuf)
                    pltpu.sync_copy(rowbuf, out.at[pl.ds(start, fbuf)])

                return carry

            lax.fori_loop(0, ngrp // 2, body, 0)

    return _sc_gather


_HALF = _B // 2
_sc_gather_half = _make_sc_gather(_HALF)


def _dense_body(gu, gi, xum, xim, gen, gW, gb, W1, b1, W2, b2, Wf, bf, out):
    xg = jnp.dot(gen[...], gW[...], preferred_element_type=jnp.float32) + gb[...]
    h = jnp.concatenate([xum[...], xim[...], xg], axis=1)
    h = jnp.maximum(
        jnp.dot(h, W1[...], preferred_element_type=jnp.float32) + b1[...], 0.0)
    h = jnp.maximum(
        jnp.dot(h, W2[...], preferred_element_type=jnp.float32) + b2[...], 0.0)
    wf = Wf[...]
    x_gmf = gu[...] * gi[...]
    acc = jnp.dot(x_gmf, wf[0:_EMB, :], preferred_element_type=jnp.float32)
    acc = acc + jnp.dot(h, wf[_EMB:, :], preferred_element_type=jnp.float32)
    out[...] = acc + bf[...]


_BT = 2048


def _dense(gu, gi, xum, xim, gen, gW, gb, W1, b1, W2, b2, Wf, bf):
    b = gu.shape[0]
    grid = (b // _BT,)
    row = lambda i: (i, 0)
    full = lambda i: (0, 0)
    return pl.pallas_call(
        _dense_body,
        grid=grid,
        in_specs=[
            pl.BlockSpec((_BT, _EMB), row),
            pl.BlockSpec((_BT, _EMB), row),
            pl.BlockSpec((_BT, _EMB), row),
            pl.BlockSpec((_BT, _EMB), row),
            pl.BlockSpec((_BT, 18), row),
            pl.BlockSpec((18, 16), full),
            pl.BlockSpec((1, 16), full),
            pl.BlockSpec((48, 128), full),
            pl.BlockSpec((1, 128), full),
            pl.BlockSpec((128, 64), full),
            pl.BlockSpec((1, 64), full),
            pl.BlockSpec((80, 1), full),
            pl.BlockSpec((1, 1), full),
        ],
        out_specs=pl.BlockSpec((_BT, 1), row),
        out_shape=jax.ShapeDtypeStruct((b, 1), jnp.float32),
        compiler_params=pltpu.CompilerParams(
            dimension_semantics=("parallel",)),
    )(gu, gi, xum, xim, gen, gW, gb, W1, b1, W2, b2, Wf, bf)


def kernel(user_indices, item_indices, genres_vec, gmf_user_emb, gmf_item_emb,
           mlp_user_emb, mlp_item_emb, genres_W, genres_b, W1, b1, W2, b2,
           Wf, bf):
    ui = user_indices.astype(jnp.int32)
    ii = item_indices.astype(jnp.int32)
    tables = (gmf_user_emb.T, gmf_item_emb.T, mlp_user_emb.T,
              mlp_item_emb.T)
    dense_rest = (genres_W, genres_b.reshape(1, -1), W1, b1.reshape(1, -1),
                  W2, b2.reshape(1, -1), Wf, bf.reshape(1, -1))
    outs = []
    for h in range(2):
        sl = slice(h * _HALF, (h + 1) * _HALF)
        rows = _sc_gather_half(
            ui[sl].reshape(_NW, _HALF // _NW),
            ii[sl].reshape(_NW, _HALF // _NW), *tables)
        outs.append(_dense(*rows, genres_vec[sl], *dense_rest))
    return jnp.concatenate(outs)[:, 0]


# speedup vs baseline: 1.0274x; 1.0274x over previous
import functools

import jax
import jax.numpy as jnp
from jax import lax
from jax.experimental import pallas as pl
from jax.experimental.pallas import tpu as pltpu
from jax.experimental.pallas import tpu_sc as plsc

_B = 16384
_EMB = 16
_NC = 2
_NS = 16
_NW = _NC * _NS

_mesh = plsc.VectorSubcoreMesh(core_axis_name="c", subcore_axis_name="s")


def _make_sc_gather(b):
    bpw = b // _NW
    ngrp = bpw // 16
    fbuf = min(256, bpw)
    fmask = fbuf // 16 - 1

    @functools.partial(
        pl.kernel,
        mesh=_mesh,
        out_type=[jax.ShapeDtypeStruct((b, _EMB), jnp.float32)] * 4,
        scratch_types=[
            pltpu.VMEM((bpw,), jnp.int32),
            pltpu.VMEM((bpw,), jnp.int32),
            [pltpu.VMEM((_EMB, 128), jnp.float32)] * 32,
            pltpu.VMEM((fbuf, _EMB), jnp.float32),
            [pltpu.SemaphoreType.DMA] * 2,
        ],
        compiler_params=pltpu.CompilerParams(needs_layout_passes=False),
    )
    def _sc_gather(uidx_hbm, iidx_hbm, gu_hbm, gi_hbm, mu_hbm, mi_hbm,
                   gu_out, gi_out, mu_out, mi_out,
                   uidx_v, iidx_v, stages, rowbuf, sems):
        wid = lax.axis_index("s") * _NC + lax.axis_index("c")
        base = wid * bpw

        pltpu.sync_copy(uidx_hbm.at[wid], uidx_v)
        pltpu.sync_copy(iidx_hbm.at[wid], iidx_v)

        lanes = lax.iota(jnp.int32, 16)

        for table, idx_v, out in (
            (gu_hbm, uidx_v, gu_out),
            (gi_hbm, iidx_v, gi_out),
            (mu_hbm, uidx_v, mu_out),
            (mi_hbm, iidx_v, mi_out),
        ):
            def fire(g, ring, table=table, idx_v=idx_v):
                vec = idx_v[pl.ds(g * 16, 16)]
                for j in range(16):
                    u = vec[j]
                    bs = pl.multiple_of((u >> 7) * 128, 128)
                    pltpu.async_copy(
                        table.at[:, pl.ds(bs, 128)], stages[ring * 16 + j],
                        sems[ring])

            def extract(g, ring, table=table, idx_v=idx_v):
                for j in range(16):
                    pltpu.make_async_copy(
                        table.at[:, pl.ds(0, 128)], stages[ring * 16 + j],
                        sems[ring]).wait()
                vec = idx_v[pl.ds(g * 16, 16)]
                for j in range(16):
                    c = vec[j] & 127
                    val = plsc.load_gather(
                        stages[ring * 16 + j],
                        [lanes, jnp.zeros((16,), jnp.int32) + c])
                    rowbuf[(g & fmask) * 16 + j, :] = val

            fire(0, 0)
            fire(1, 1)

            def body(h, carry, out=out, fire=fire, extract=extract):
                g0 = 2 * h
                g1 = 2 * h + 1
                extract(g0, 0)

                @pl.when(g0 + 2 < ngrp)
                def _():
                    fire(g0 + 2, 0)

                extract(g1, 1)

                @pl.when(g1 + 2 < ngrp)
                def _():
                    fire(g1 + 2, 1)

                @pl.when((g1 & fmask) == fmask)
                def _():
                    start = pl.multiple_of(
                        base + (g1 // (fbuf // 16)) * fbuf, fbuf)
                    pltpu.sync_copy(rowbuf, out.at[pl.ds(start, fbuf)])

                return carry

            lax.fori_loop(0, ngrp // 2, body, 0)

    return _sc_gather


_HALF = _B
_sc_gather_half = _make_sc_gather(_HALF)


def _dense_body(gu, gi, xum, xim, gen, gW, gb, W1, b1, W2, b2, Wf, bf, out):
    xg = jnp.dot(gen[...], gW[...], preferred_element_type=jnp.float32) + gb[...]
    h = jnp.concatenate([xum[...], xim[...], xg], axis=1)
    h = jnp.maximum(
        jnp.dot(h, W1[...], preferred_element_type=jnp.float32) + b1[...], 0.0)
    h = jnp.maximum(
        jnp.dot(h, W2[...], preferred_element_type=jnp.float32) + b2[...], 0.0)
    wf = Wf[...]
    x_gmf = gu[...] * gi[...]
    acc = jnp.dot(x_gmf, wf[0:_EMB, :], preferred_element_type=jnp.float32)
    acc = acc + jnp.dot(h, wf[_EMB:, :], preferred_element_type=jnp.float32)
    out[...] = acc + bf[...]


_BT = 2048


def _dense(gu, gi, xum, xim, gen, gW, gb, W1, b1, W2, b2, Wf, bf):
    b = gu.shape[0]
    grid = (b // _BT,)
    row = lambda i: (i, 0)
    full = lambda i: (0, 0)
    return pl.pallas_call(
        _dense_body,
        grid=grid,
        in_specs=[
            pl.BlockSpec((_BT, _EMB), row),
            pl.BlockSpec((_BT, _EMB), row),
            pl.BlockSpec((_BT, _EMB), row),
            pl.BlockSpec((_BT, _EMB), row),
            pl.BlockSpec((_BT, 18), row),
            pl.BlockSpec((18, 16), full),
            pl.BlockSpec((1, 16), full),
            pl.BlockSpec((48, 128), full),
            pl.BlockSpec((1, 128), full),
            pl.BlockSpec((128, 64), full),
            pl.BlockSpec((1, 64), full),
            pl.BlockSpec((80, 1), full),
            pl.BlockSpec((1, 1), full),
        ],
        out_specs=pl.BlockSpec((_BT, 1), row),
        out_shape=jax.ShapeDtypeStruct((b, 1), jnp.float32),
        compiler_params=pltpu.CompilerParams(
            dimension_semantics=("parallel",)),
    )(gu, gi, xum, xim, gen, gW, gb, W1, b1, W2, b2, Wf, bf)


def kernel(user_indices, item_indices, genres_vec, gmf_user_emb, gmf_item_emb,
           mlp_user_emb, mlp_item_emb, genres_W, genres_b, W1, b1, W2, b2,
           Wf, bf):
    ui = user_indices.astype(jnp.int32)
    ii = item_indices.astype(jnp.int32)
    tables = (gmf_user_emb.T, gmf_item_emb.T, mlp_user_emb.T,
              mlp_item_emb.T)
    dense_rest = (genres_W, genres_b.reshape(1, -1), W1, b1.reshape(1, -1),
                  W2, b2.reshape(1, -1), Wf, bf.reshape(1, -1))
    outs = []
    for h in range(_B // _HALF):
        sl = slice(h * _HALF, (h + 1) * _HALF)
        rows = _sc_gather_half(
            ui[sl].reshape(_NW, _HALF // _NW),
            ii[sl].reshape(_NW, _HALF // _NW), *tables)
        outs.append(_dense(*rows, genres_vec[sl], *dense_rest))
    return jnp.concatenate(outs)[:, 0]


# speedup vs baseline: 1.1155x; 1.0858x over previous
import functools

import jax
import jax.numpy as jnp
from jax import lax
from jax.experimental import pallas as pl
from jax.experimental.pallas import tpu as pltpu
from jax.experimental.pallas import tpu_sc as plsc

_B = 16384
_EMB = 16
_NC = 2
_NS = 16
_NW = _NC * _NS

_mesh = plsc.VectorSubcoreMesh(core_axis_name="c", subcore_axis_name="s")


def _make_sc_gather(b):
    bpw = b // _NW
    ngrp = bpw // 16
    fbuf = min(128, bpw)
    fmask = fbuf // 16 - 1
    nring = 3

    @functools.partial(
        pl.kernel,
        mesh=_mesh,
        out_type=[jax.ShapeDtypeStruct((b, _EMB), jnp.float32)] * 4,
        scratch_types=[
            pltpu.VMEM((bpw,), jnp.int32),
            pltpu.VMEM((bpw,), jnp.int32),
            [pltpu.VMEM((_EMB, 128), jnp.float32)] * 48,
            pltpu.VMEM((fbuf, _EMB), jnp.float32),
            [pltpu.SemaphoreType.DMA] * 3,
        ],
        compiler_params=pltpu.CompilerParams(needs_layout_passes=False),
    )
    def _sc_gather(uidx_hbm, iidx_hbm, gu_hbm, gi_hbm, mu_hbm, mi_hbm,
                   gu_out, gi_out, mu_out, mi_out,
                   uidx_v, iidx_v, stages, rowbuf, sems):
        wid = lax.axis_index("s") * _NC + lax.axis_index("c")
        base = wid * bpw

        pltpu.sync_copy(uidx_hbm.at[wid], uidx_v)
        pltpu.sync_copy(iidx_hbm.at[wid], iidx_v)

        lanes = lax.iota(jnp.int32, 16)

        for table, idx_v, out in (
            (gu_hbm, uidx_v, gu_out),
            (gi_hbm, iidx_v, gi_out),
            (mu_hbm, uidx_v, mu_out),
            (mi_hbm, iidx_v, mi_out),
        ):
            def fire(g, ring, table=table, idx_v=idx_v):
                vec = idx_v[pl.ds(g * 16, 16)]
                for j in range(16):
                    u = vec[j]
                    bs = pl.multiple_of((u >> 7) * 128, 128)
                    pltpu.async_copy(
                        table.at[:, pl.ds(bs, 128)], stages[ring * 16 + j],
                        sems[ring])

            def extract(g, ring, table=table, idx_v=idx_v):
                for j in range(16):
                    pltpu.make_async_copy(
                        table.at[:, pl.ds(0, 128)], stages[ring * 16 + j],
                        sems[ring]).wait()
                vec = idx_v[pl.ds(g * 16, 16)]
                for j in range(16):
                    c = vec[j] & 127
                    val = plsc.load_gather(
                        stages[ring * 16 + j],
                        [lanes, jnp.zeros((16,), jnp.int32) + c])
                    rowbuf[(g & fmask) * 16 + j, :] = val

            for r in range(nring):
                fire(r, r)

            def body(h, carry, out=out, fire=fire, extract=extract):
                for k in range(nring):
                    g = nring * h + k

                    @pl.when(g < ngrp)
                    def _(g=g, k=k):
                        extract(g, k)

                        @pl.when(g + nring < ngrp)
                        def _():
                            fire(g + nring, k)

                        @pl.when((g & fmask) == fmask)
                        def _():
                            start = pl.multiple_of(
                                base + (g // (fbuf // 16)) * fbuf, fbuf)
                            pltpu.sync_copy(rowbuf, out.at[pl.ds(start, fbuf)])

                return carry

            lax.fori_loop(0, (ngrp + nring - 1) // nring, body, 0)

    return _sc_gather


_HALF = _B
_sc_gather_half = _make_sc_gather(_HALF)


def _dense_body(gu, gi, xum, xim, gen, gW, gb, W1, b1, W2, b2, Wf, bf, out):
    xg = jnp.dot(gen[...], gW[...], preferred_element_type=jnp.float32) + gb[...]
    h = jnp.concatenate([xum[...], xim[...], xg], axis=1)
    h = jnp.maximum(
        jnp.dot(h, W1[...], preferred_element_type=jnp.float32) + b1[...], 0.0)
    h = jnp.maximum(
        jnp.dot(h, W2[...], preferred_element_type=jnp.float32) + b2[...], 0.0)
    wf = Wf[...]
    x_gmf = gu[...] * gi[...]
    acc = jnp.dot(x_gmf, wf[0:_EMB, :], preferred_element_type=jnp.float32)
    acc = acc + jnp.dot(h, wf[_EMB:, :], preferred_element_type=jnp.float32)
    out[...] = acc + bf[...]


_BT = 2048


def _dense(gu, gi, xum, xim, gen, gW, gb, W1, b1, W2, b2, Wf, bf):
    b = gu.shape[0]
    grid = (b // _BT,)
    row = lambda i: (i, 0)
    full = lambda i: (0, 0)
    return pl.pallas_call(
        _dense_body,
        grid=grid,
        in_specs=[
            pl.BlockSpec((_BT, _EMB), row),
            pl.BlockSpec((_BT, _EMB), row),
            pl.BlockSpec((_BT, _EMB), row),
            pl.BlockSpec((_BT, _EMB), row),
            pl.BlockSpec((_BT, 18), row),
            pl.BlockSpec((18, 16), full),
            pl.BlockSpec((1, 16), full),
            pl.BlockSpec((48, 128), full),
            pl.BlockSpec((1, 128), full),
            pl.BlockSpec((128, 64), full),
            pl.BlockSpec((1, 64), full),
            pl.BlockSpec((80, 1), full),
            pl.BlockSpec((1, 1), full),
        ],
        out_specs=pl.BlockSpec((_BT, 1), row),
        out_shape=jax.ShapeDtypeStruct((b, 1), jnp.float32),
        compiler_params=pltpu.CompilerParams(
            dimension_semantics=("parallel",)),
    )(gu, gi, xum, xim, gen, gW, gb, W1, b1, W2, b2, Wf, bf)


def kernel(user_indices, item_indices, genres_vec, gmf_user_emb, gmf_item_emb,
           mlp_user_emb, mlp_item_emb, genres_W, genres_b, W1, b1, W2, b2,
           Wf, bf):
    ui = user_indices.astype(jnp.int32)
    ii = item_indices.astype(jnp.int32)
    tables = (gmf_user_emb.T, gmf_item_emb.T, mlp_user_emb.T,
              mlp_item_emb.T)
    dense_rest = (genres_W, genres_b.reshape(1, -1), W1, b1.reshape(1, -1),
                  W2, b2.reshape(1, -1), Wf, bf.reshape(1, -1))
    outs = []
    for h in range(_B // _HALF):
        sl = slice(h * _HALF, (h + 1) * _HALF)
        rows = _sc_gather_half(
            ui[sl].reshape(_NW, _HALF // _NW),
            ii[sl].reshape(_NW, _HALF // _NW), *tables)
        outs.append(_dense(*rows, genres_vec[sl], *dense_rest))
    return jnp.concatenate(outs)[:, 0]


# speedup vs baseline: 1.1949x; 1.0712x over previous
import functools

import jax
import jax.numpy as jnp
from jax import lax
from jax.experimental import pallas as pl
from jax.experimental.pallas import tpu as pltpu
from jax.experimental.pallas import tpu_sc as plsc

_B = 16384
_EMB = 16
_NC = 2
_NS = 16
_NW = _NC * _NS

_mesh = plsc.VectorSubcoreMesh(core_axis_name="c", subcore_axis_name="s")


def _make_sc_gather(b):
    bpw = b // _NW
    ngrp = bpw // 16
    fbuf = min(128, bpw)
    fmask = fbuf // 16 - 1
    nring = 3

    @functools.partial(
        pl.kernel,
        mesh=_mesh,
        out_type=[jax.ShapeDtypeStruct((_EMB, b), jnp.float32)] * 4,
        scratch_types=[
            pltpu.VMEM((bpw,), jnp.int32),
            pltpu.VMEM((bpw,), jnp.int32),
            [pltpu.VMEM((_EMB, 128), jnp.float32)] * 48,
            pltpu.VMEM((_EMB, fbuf), jnp.float32),
            [pltpu.SemaphoreType.DMA] * 3,
        ],
        compiler_params=pltpu.CompilerParams(needs_layout_passes=False),
    )
    def _sc_gather(uidx_hbm, iidx_hbm, gu_hbm, gi_hbm, mu_hbm, mi_hbm,
                   gu_out, gi_out, mu_out, mi_out,
                   uidx_v, iidx_v, stages, rowbuf, sems):
        wid = lax.axis_index("s") * _NC + lax.axis_index("c")
        base = wid * bpw

        pltpu.sync_copy(uidx_hbm.at[wid], uidx_v)
        pltpu.sync_copy(iidx_hbm.at[wid], iidx_v)

        lanes = lax.iota(jnp.int32, 16)

        for table, idx_v, out in (
            (gu_hbm, uidx_v, gu_out),
            (gi_hbm, iidx_v, gi_out),
            (mu_hbm, uidx_v, mu_out),
            (mi_hbm, iidx_v, mi_out),
        ):
            def fire(g, ring, table=table, idx_v=idx_v):
                vec = idx_v[pl.ds(g * 16, 16)]
                for j in range(16):
                    u = vec[j]
                    bs = pl.multiple_of((u >> 7) * 128, 128)
                    pltpu.async_copy(
                        table.at[:, pl.ds(bs, 128)], stages[ring * 16 + j],
                        sems[ring])

            def extract(g, ring, table=table, idx_v=idx_v):
                for j in range(16):
                    pltpu.make_async_copy(
                        table.at[:, pl.ds(0, 128)], stages[ring * 16 + j],
                        sems[ring]).wait()
                vec = idx_v[pl.ds(g * 16, 16)]
                for j in range(16):
                    c = vec[j] & 127
                    val = plsc.load_gather(
                        stages[ring * 16 + j],
                        [lanes, jnp.zeros((16,), jnp.int32) + c])
                    pos = (g & fmask) * 16 + j
                    plsc.store_scatter(
                        rowbuf, [lanes, jnp.zeros((16,), jnp.int32) + pos],
                        val)

            for r in range(nring):
                fire(r, r)

            def body(h, carry, out=out, fire=fire, extract=extract):
                for k in range(nring):
                    g = nring * h + k

                    @pl.when(g < ngrp)
                    def _(g=g, k=k):
                        extract(g, k)

                        @pl.when(g + nring < ngrp)
                        def _():
                            fire(g + nring, k)

                        @pl.when((g & fmask) == fmask)
                        def _():
                            start = pl.multiple_of(
                                base + (g // (fbuf // 16)) * fbuf, fbuf)
                            pltpu.sync_copy(
                                rowbuf, out.at[:, pl.ds(start, fbuf)])

                return carry

            lax.fori_loop(0, (ngrp + nring - 1) // nring, body, 0)

    return _sc_gather


_HALF = _B
_sc_gather_half = _make_sc_gather(_HALF)


def _dense_body(gu, gi, xum, xim, gen, gWT, gb, W1T, b1, W2T, b2, WfT, bf,
                out):
    xg = jnp.dot(gWT[...], gen[...], preferred_element_type=jnp.float32) + gb[...]
    h = jnp.concatenate([xum[...], xim[...], xg], axis=0)
    h = jnp.maximum(
        jnp.dot(W1T[...], h, preferred_element_type=jnp.float32) + b1[...], 0.0)
    h = jnp.maximum(
        jnp.dot(W2T[...], h, preferred_element_type=jnp.float32) + b2[...], 0.0)
    wf = WfT[...]
    x_gmf = gu[...] * gi[...]
    acc = jnp.dot(wf[:, 0:_EMB], x_gmf, preferred_element_type=jnp.float32)
    acc = acc + jnp.dot(wf[:, _EMB:], h, preferred_element_type=jnp.float32)
    out[...] = acc + bf[...]


_BT = 2048


def _dense(gu, gi, xum, xim, gen, gWT, gb, W1T, b1, W2T, b2, WfT, bf):
    b = gu.shape[1]
    grid = (b // _BT,)
    col = lambda i: (0, i)
    full = lambda i: (0, 0)
    return pl.pallas_call(
        _dense_body,
        grid=grid,
        in_specs=[
            pl.BlockSpec((_EMB, _BT), col),
            pl.BlockSpec((_EMB, _BT), col),
            pl.BlockSpec((_EMB, _BT), col),
            pl.BlockSpec((_EMB, _BT), col),
            pl.BlockSpec((18, _BT), col),
            pl.BlockSpec((16, 18), full),
            pl.BlockSpec((16, 1), full),
            pl.BlockSpec((128, 48), full),
            pl.BlockSpec((128, 1), full),
            pl.BlockSpec((64, 128), full),
            pl.BlockSpec((64, 1), full),
            pl.BlockSpec((1, 80), full),
            pl.BlockSpec((1, 1), full),
        ],
        out_specs=pl.BlockSpec((1, _BT), col),
        out_shape=jax.ShapeDtypeStruct((1, b), jnp.float32),
        compiler_params=pltpu.CompilerParams(
            dimension_semantics=("parallel",)),
    )(gu, gi, xum, xim, gen, gWT, gb, W1T, b1, W2T, b2, WfT, bf)


def kernel(user_indices, item_indices, genres_vec, gmf_user_emb, gmf_item_emb,
           mlp_user_emb, mlp_item_emb, genres_W, genres_b, W1, b1, W2, b2,
           Wf, bf):
    ui = user_indices.astype(jnp.int32)
    ii = item_indices.astype(jnp.int32)
    tables = (gmf_user_emb.T, gmf_item_emb.T, mlp_user_emb.T,
              mlp_item_emb.T)
    dense_rest = (genres_W.T, genres_b.reshape(-1, 1), W1.T,
                  b1.reshape(-1, 1), W2.T, b2.reshape(-1, 1), Wf.T,
                  bf.reshape(-1, 1))
    rows = _sc_gather_half(
        ui.reshape(_NW, _B // _NW), ii.reshape(_NW, _B // _NW), *tables)
    out = _dense(*rows, genres_vec.T, *dense_rest)
    return out[0, :]
